# baseline (speedup 1.0000x reference)
import jax
import jax.numpy as jnp
from jax import lax
from jax.experimental import pallas as pl
from jax.experimental.pallas import tpu as pltpu
from jax.experimental.pallas import tpu_sc as plsc

N = 10000
E = 320000
C = 128
K = 80
CHUNK = 16
DCHUNK = 16
NSUB = 16
RPS = 624
RPS_LAST = N - RPS * (NSUB - 1)
EP = 327680
NPAD = 10016
ROWS = EP // K
RB = 1000

_mesh = plsc.VectorSubcoreMesh(
    core_axis_name="c", subcore_axis_name="s", num_cores=2, num_subcores=NSUB)


def _part_copy(sid, mk_src, mk_dst):
    base = pl.multiple_of(sid * RPS, 8)
    @pl.when(sid < NSUB - 1)
    def _():
        pltpu.sync_copy(mk_src(pl.ds(base, RPS)), mk_dst(pl.ds(base, RPS)))
    @pl.when(sid == NSUB - 1)
    def _():
        last = pl.ds((NSUB - 1) * RPS, RPS_LAST)
        pltpu.sync_copy(mk_src(last), mk_dst(last))


def _deg_body(dst_hbm, ones_hbm, out_hbm, cnt_sh, ones_v, dchunk, sem):
    cid = lax.axis_index("c")
    sid = lax.axis_index("s")
    _part_copy(sid, lambda s: ones_hbm.at[pl.ds(0, s.size)], lambda s: cnt_sh.at[s])
    pltpu.sync_copy(ones_hbm.at[pl.ds(0, K)], ones_v)
    plsc.subcore_barrier()
    wchunk = (cid * NSUB + sid) * (ROWS // 32 // DCHUNK)
    @pl.loop(0, (ROWS // 32) // DCHUNK)
    def _(ci):
        pltpu.sync_copy(dst_hbm.at[wchunk + ci], dchunk)
        @pl.loop(0, DCHUNK)
        def _(j):
            pltpu.sync_copy(ones_v, cnt_sh.at[dchunk.at[j]], add=True)
    plsc.subcore_barrier()
    @pl.when(cid == 0)
    def _():
        _part_copy(sid, lambda s: cnt_sh.at[s], lambda s: out_hbm.at[0].at[s])
    @pl.when(cid == 1)
    def _():
        _part_copy(sid, lambda s: cnt_sh.at[s], lambda s: out_hbm.at[1].at[s])


_deg = pl.kernel(
    _deg_body,
    out_type=jax.ShapeDtypeStruct((2, N, 16), jnp.float32),
    mesh=_mesh,
    scratch_types=[
        pltpu.VMEM_SHARED((NPAD, 16), jnp.float32),
        pltpu.VMEM((K, 16), jnp.float32),
        pltpu.VMEM((DCHUNK, K), jnp.int32),
        pltpu.SemaphoreType.DMA,
    ],
)


def _agg_body(ha_hbm, hb_hbm, src_hbm, dst_hbm, out_hbm,
              acc_sh, schunk, dchunk, rows_a, rows_b, rows_c, rows_d, sem):
    cid = lax.axis_index("c")
    sid = lax.axis_index("s")
    bufs = (rows_a, rows_b, rows_c, rows_d)

    def run(h_hbm, out_slot):
        _part_copy(sid, lambda s: h_hbm.at[s], lambda s: acc_sh.at[s])
        plsc.subcore_barrier()
        base = sid * (ROWS // NSUB // CHUNK)
        @pl.loop(0, (ROWS // NSUB) // CHUNK)
        def _(ci):
            pltpu.sync_copy(src_hbm.at[base + ci], schunk)
            pltpu.sync_copy(dst_hbm.at[base + ci], dchunk)
            descs = [None] * CHUNK

            def gather(j):
                descs[j] = pltpu.async_copy(
                    h_hbm.at[schunk.at[j]], bufs[j % 4], sem)

            gather(0)
            gather(1)
            gather(2)
            for j in range(CHUNK):
                if j + 3 < CHUNK:
                    gather(j + 3)
                descs[j].wait()
                pltpu.sync_copy(bufs[j % 4], acc_sh.at[dchunk.at[j]],
                                add=True)
        plsc.subcore_barrier()
        _part_copy(sid, lambda s: acc_sh.at[s], lambda s: out_slot.at[s])

    @pl.when(cid == 0)
    def _():
        run(ha_hbm, out_hbm.at[0])
    @pl.when(cid == 1)
    def _():
        run(hb_hbm, out_hbm.at[1])


_agg = pl.kernel(
    _agg_body,
    out_type=jax.ShapeDtypeStruct((2, N, C), jnp.float32),
    mesh=_mesh,
    scratch_types=[
        pltpu.VMEM_SHARED((NPAD, C), jnp.float32),
        pltpu.VMEM((CHUNK, K), jnp.int32),
        pltpu.VMEM((CHUNK, K), jnp.int32),
        pltpu.VMEM((K, C), jnp.float32),
        pltpu.VMEM((K, C), jnp.float32),
        pltpu.VMEM((K, C), jnp.float32),
        pltpu.VMEM((K, C), jnp.float32),
        pltpu.SemaphoreType.DMA,
    ],
)


def _dinv_of(p_ref):
    deg = p_ref[0, :, 0] + p_ref[1, :, 0] - 1.0
    return lax.rsqrt(deg)[:, None]


def _mm1_body(x_ref, w_ref, o_ref):
    o_ref[...] = jnp.dot(x_ref[...], w_ref[...],
                         preferred_element_type=jnp.float32)


def _scale_body(u_ref, p_ref, a_ref, b_ref):
    dinv = _dinv_of(p_ref)
    a_ref[...] = u_ref[:, :C] * dinv
    b_ref[...] = u_ref[:, C:] * dinv


def _mid_body(agg_ref, p_ref, b1_ref, wmu_ref, wls_ref, omu_ref, ols_ref):
    dinv = _dinv_of(p_ref)
    h = jnp.concatenate([agg_ref[0], agg_ref[1]], axis=1) * dinv + b1_ref[...]
    nrm = jnp.sqrt(jnp.sum(h * h, axis=1, keepdims=True))
    h = jnp.maximum(h / jnp.maximum(nrm, 1e-12), 0.0)
    omu_ref[...] = jnp.dot(h, wmu_ref[...],
                           preferred_element_type=jnp.float32) * dinv
    ols_ref[...] = jnp.dot(h, wls_ref[...],
                           preferred_element_type=jnp.float32) * dinv


def _fin_body(agg_ref, p_ref, bmu_ref, bls_ref, mu_ref, ls_ref):
    dinv = _dinv_of(p_ref)
    mu_ref[...] = agg_ref[0] * dinv + bmu_ref[...]
    ls_ref[...] = agg_ref[1] * dinv + bls_ref[...]


def _row_spec(width):
    return pl.BlockSpec((RB, width), lambda i: (i, 0))


_P_SPEC = pl.BlockSpec((2, RB, 16), lambda i: (0, i, 0))
_AGG_SPEC = pl.BlockSpec((2, RB, C), lambda i: (0, i, 0))


def _full(shape):
    return pl.BlockSpec(shape, lambda i: tuple(0 for _ in shape))


def kernel(x, edge_index, W1, b1, W_mu, b_mu, W_ls, b_ls):
    srcf = edge_index[0].astype(jnp.int32)
    dstf = edge_index[1].astype(jnp.int32)
    srcp = jnp.concatenate([srcf, jnp.zeros((EP - E,), jnp.int32)])
    dstp = jnp.concatenate([dstf, jnp.full((EP - E,), N, jnp.int32)])
    src = srcp.reshape(ROWS // CHUNK, CHUNK, K)
    dst = dstp.reshape(ROWS // CHUNK, CHUNK, K)
    dstd = dstp.reshape(ROWS // DCHUNK, DCHUNK, K)
    ones16 = jnp.ones((RPS_LAST, 16), jnp.float32)
    grid = (N // RB,)
    f32 = jnp.float32

    p = _deg(dstd, ones16)
    u = pl.pallas_call(
        _mm1_body, grid=grid,
        in_specs=[_row_spec(C), _full((C, 2 * C))],
        out_specs=_row_spec(2 * C),
        out_shape=jax.ShapeDtypeStruct((N, 2 * C), f32))(x, W1)
    ha, hb = pl.pallas_call(
        _scale_body, grid=grid,
        in_specs=[_row_spec(2 * C), _P_SPEC],
        out_specs=[_row_spec(C), _row_spec(C)],
        out_shape=[jax.ShapeDtypeStruct((N, C), f32)] * 2)(u, p)
    agg1 = _agg(ha, hb, src, dst)
    hmu, hls = pl.pallas_call(
        _mid_body, grid=grid,
        in_specs=[_AGG_SPEC, _P_SPEC, _full((1, 2 * C)),
                  _full((2 * C, C)), _full((2 * C, C))],
        out_specs=[_row_spec(C), _row_spec(C)],
        out_shape=[jax.ShapeDtypeStruct((N, C), f32)] * 2)(
            agg1, p, b1.reshape(1, 2 * C), W_mu, W_ls)
    agg2 = _agg(hmu, hls, src, dst)
    mu, ls = pl.pallas_call(
        _fin_body, grid=grid,
        in_specs=[_AGG_SPEC, _P_SPEC, _full((1, C)), _full((1, C))],
        out_specs=[_row_spec(C), _row_spec(C)],
        out_shape=[jax.ShapeDtypeStruct((N, C), f32)] * 2)(
            agg2, p, b_mu.reshape(1, C), b_ls.reshape(1, C))
    return (mu, ls)


# speedup vs baseline: 2.3863x; 2.3863x over previous
import jax
import jax.numpy as jnp
from jax import lax
from jax.experimental import pallas as pl
from jax.experimental.pallas import tpu as pltpu
from jax.experimental.pallas import tpu_sc as plsc

N = 10000
E = 320000
C = 128
K = 80
CHUNK = 10
DCHUNK = 25
NSUB = 16
RPS = 624
RPS_LAST = N - RPS * (NSUB - 1)
ROWS = E // K
RB = 1000

_mesh = plsc.VectorSubcoreMesh(
    core_axis_name="c", subcore_axis_name="s", num_cores=2, num_subcores=NSUB)


def _part_copy(sid, mk_src, mk_dst):
    base = pl.multiple_of(sid * RPS, 8)
    @pl.when(sid < NSUB - 1)
    def _():
        pltpu.sync_copy(mk_src(pl.ds(base, RPS)), mk_dst(pl.ds(base, RPS)))
    @pl.when(sid == NSUB - 1)
    def _():
        last = pl.ds((NSUB - 1) * RPS, RPS_LAST)
        pltpu.sync_copy(mk_src(last), mk_dst(last))


def _deg_body(dst_hbm, ones_hbm, out_hbm, cnt_sh, ones_v, dchunk, sem):
    cid = lax.axis_index("c")
    sid = lax.axis_index("s")
    _part_copy(sid, lambda s: ones_hbm.at[pl.ds(0, s.size)], lambda s: cnt_sh.at[s])
    pltpu.sync_copy(ones_hbm.at[pl.ds(0, K)], ones_v)
    plsc.subcore_barrier()
    wchunk = (cid * NSUB + sid) * (ROWS // 32 // DCHUNK)
    @pl.loop(0, (ROWS // 32) // DCHUNK)
    def _(ci):
        pltpu.sync_copy(dst_hbm.at[wchunk + ci], dchunk)
        @pl.loop(0, DCHUNK)
        def _(j):
            pltpu.sync_copy(ones_v, cnt_sh.at[dchunk.at[j]], add=True)
    plsc.subcore_barrier()
    @pl.when(cid == 0)
    def _():
        _part_copy(sid, lambda s: cnt_sh.at[s], lambda s: out_hbm.at[0].at[s])
    @pl.when(cid == 1)
    def _():
        _part_copy(sid, lambda s: cnt_sh.at[s], lambda s: out_hbm.at[1].at[s])


_deg = pl.kernel(
    _deg_body,
    out_type=jax.ShapeDtypeStruct((2, N, 16), jnp.float32),
    mesh=_mesh,
    scratch_types=[
        pltpu.VMEM_SHARED((N, 16), jnp.float32),
        pltpu.VMEM((K, 16), jnp.float32),
        pltpu.VMEM((DCHUNK, K), jnp.int32),
        pltpu.SemaphoreType.DMA,
    ],
)


def _agg_body(ha_hbm, hb_hbm, src_hbm, dst_hbm, out_hbm,
              acc_sh, schunk, dchunk, rows_a, rows_b, rows_c, rows_d, sem):
    cid = lax.axis_index("c")
    sid = lax.axis_index("s")
    bufs = (rows_a, rows_b, rows_c, rows_d)

    def run(h_hbm, out_slot):
        _part_copy(sid, lambda s: h_hbm.at[s], lambda s: acc_sh.at[s])
        plsc.subcore_barrier()
        base = sid * (ROWS // NSUB // CHUNK)
        @pl.loop(0, (ROWS // NSUB) // CHUNK)
        def _(ci):
            pltpu.sync_copy(src_hbm.at[base + ci], schunk)
            pltpu.sync_copy(dst_hbm.at[base + ci], dchunk)
            descs = [None] * CHUNK

            def gather(j):
                descs[j] = pltpu.async_copy(
                    h_hbm.at[schunk.at[j]], bufs[j % 4], sem)

            gather(0)
            gather(1)
            gather(2)
            for j in range(CHUNK):
                if j + 3 < CHUNK:
                    gather(j + 3)
                descs[j].wait()
                pltpu.sync_copy(bufs[j % 4], acc_sh.at[dchunk.at[j]],
                                add=True)
        plsc.subcore_barrier()
        _part_copy(sid, lambda s: acc_sh.at[s], lambda s: out_slot.at[s])

    @pl.when(cid == 0)
    def _():
        run(ha_hbm, out_hbm.at[0])
    @pl.when(cid == 1)
    def _():
        run(hb_hbm, out_hbm.at[1])


_agg = pl.kernel(
    _agg_body,
    out_type=jax.ShapeDtypeStruct((2, N, C), jnp.float32),
    mesh=_mesh,
    scratch_types=[
        pltpu.VMEM_SHARED((N, C), jnp.float32),
        pltpu.VMEM((CHUNK, K), jnp.int32),
        pltpu.VMEM((CHUNK, K), jnp.int32),
        pltpu.VMEM((K, C), jnp.float32),
        pltpu.VMEM((K, C), jnp.float32),
        pltpu.VMEM((K, C), jnp.float32),
        pltpu.VMEM((K, C), jnp.float32),
        pltpu.SemaphoreType.DMA,
    ],
)


def _dinv_of(p_ref):
    deg = p_ref[0, :, 0] + p_ref[1, :, 0] - 1.0
    return lax.rsqrt(deg)[:, None]


def _mm1_body(x_ref, w_ref, o_ref):
    o_ref[...] = jnp.dot(x_ref[...], w_ref[...],
                         preferred_element_type=jnp.float32)


def _scale_body(u_ref, p_ref, a_ref, b_ref):
    dinv = _dinv_of(p_ref)
    a_ref[...] = u_ref[:, :C] * dinv
    b_ref[...] = u_ref[:, C:] * dinv


def _mid_body(agg_ref, p_ref, b1_ref, wmu_ref, wls_ref, omu_ref, ols_ref):
    dinv = _dinv_of(p_ref)
    h = jnp.concatenate([agg_ref[0], agg_ref[1]], axis=1) * dinv + b1_ref[...]
    nrm = jnp.sqrt(jnp.sum(h * h, axis=1, keepdims=True))
    h = jnp.maximum(h / jnp.maximum(nrm, 1e-12), 0.0)
    omu_ref[...] = jnp.dot(h, wmu_ref[...],
                           preferred_element_type=jnp.float32) * dinv
    ols_ref[...] = jnp.dot(h, wls_ref[...],
                           preferred_element_type=jnp.float32) * dinv


def _fin_body(agg_ref, p_ref, bmu_ref, bls_ref, mu_ref, ls_ref):
    dinv = _dinv_of(p_ref)
    mu_ref[...] = agg_ref[0] * dinv + bmu_ref[...]
    ls_ref[...] = agg_ref[1] * dinv + bls_ref[...]


def _row_spec(width):
    return pl.BlockSpec((RB, width), lambda i: (i, 0))


_P_SPEC = pl.BlockSpec((2, RB, 16), lambda i: (0, i, 0))
_AGG_SPEC = pl.BlockSpec((2, RB, C), lambda i: (0, i, 0))


def _full(shape):
    return pl.BlockSpec(shape, lambda i: tuple(0 for _ in shape))


def kernel(x, edge_index, W1, b1, W_mu, b_mu, W_ls, b_ls):
    src = edge_index[0].astype(jnp.int32).reshape(ROWS // CHUNK, CHUNK, K)
    dst = edge_index[1].astype(jnp.int32).reshape(ROWS // CHUNK, CHUNK, K)
    dstd = edge_index[1].astype(jnp.int32).reshape(ROWS // DCHUNK, DCHUNK, K)
    ones16 = jnp.ones((RPS_LAST, 16), jnp.float32)
    grid = (N // RB,)
    f32 = jnp.float32

    p = _deg(dstd, ones16)
    u = pl.pallas_call(
        _mm1_body, grid=grid,
        in_specs=[_row_spec(C), _full((C, 2 * C))],
        out_specs=_row_spec(2 * C),
        out_shape=jax.ShapeDtypeStruct((N, 2 * C), f32))(x, W1)
    ha, hb = pl.pallas_call(
        _scale_body, grid=grid,
        in_specs=[_row_spec(2 * C), _P_SPEC],
        out_specs=[_row_spec(C), _row_spec(C)],
        out_shape=[jax.ShapeDtypeStruct((N, C), f32)] * 2)(u, p)
    agg1 = _agg(ha, hb, src, dst)
    hmu, hls = pl.pallas_call(
        _mid_body, grid=grid,
        in_specs=[_AGG_SPEC, _P_SPEC, _full((1, 2 * C)),
                  _full((2 * C, C)), _full((2 * C, C))],
        out_specs=[_row_spec(C), _row_spec(C)],
        out_shape=[jax.ShapeDtypeStruct((N, C), f32)] * 2)(
            agg1, p, b1.reshape(1, 2 * C), W_mu, W_ls)
    agg2 = _agg(hmu, hls, src, dst)
    mu, ls = pl.pallas_call(
        _fin_body, grid=grid,
        in_specs=[_AGG_SPEC, _P_SPEC, _full((1, C)), _full((1, C))],
        out_specs=[_row_spec(C), _row_spec(C)],
        out_shape=[jax.ShapeDtypeStruct((N, C), f32)] * 2)(
            agg2, p, b_mu.reshape(1, C), b_ls.reshape(1, C))
    return (mu, ls)


# speedup vs baseline: 2.6280x; 1.1013x over previous
import jax
import jax.numpy as jnp
from jax import lax
from jax.experimental import pallas as pl
from jax.experimental.pallas import tpu as pltpu
from jax.experimental.pallas import tpu_sc as plsc

N = 10000
E = 320000
C = 128
K = 80
CHUNK = 25
DCHUNK = 25
NSUB = 16
RPS = 624
RPS_LAST = N - RPS * (NSUB - 1)
ROWS = E // K
RB = 1000

_mesh = plsc.VectorSubcoreMesh(
    core_axis_name="c", subcore_axis_name="s", num_cores=2, num_subcores=NSUB)


def _part_copy(sid, mk_src, mk_dst):
    base = pl.multiple_of(sid * RPS, 8)
    @pl.when(sid < NSUB - 1)
    def _():
        pltpu.sync_copy(mk_src(pl.ds(base, RPS)), mk_dst(pl.ds(base, RPS)))
    @pl.when(sid == NSUB - 1)
    def _():
        last = pl.ds((NSUB - 1) * RPS, RPS_LAST)
        pltpu.sync_copy(mk_src(last), mk_dst(last))


def _deg_body(dst_hbm, ones_hbm, out_hbm, cnt_sh, ones_v, dchunk, sem):
    cid = lax.axis_index("c")
    sid = lax.axis_index("s")
    _part_copy(sid, lambda s: ones_hbm.at[pl.ds(0, s.size)], lambda s: cnt_sh.at[s])
    pltpu.sync_copy(ones_hbm.at[pl.ds(0, K)], ones_v)
    plsc.subcore_barrier()
    wchunk = (cid * NSUB + sid) * (ROWS // 32 // DCHUNK)
    @pl.loop(0, (ROWS // 32) // DCHUNK)
    def _(ci):
        pltpu.sync_copy(dst_hbm.at[wchunk + ci], dchunk)
        @pl.loop(0, DCHUNK)
        def _(j):
            pltpu.sync_copy(ones_v, cnt_sh.at[dchunk.at[j]], add=True)
    plsc.subcore_barrier()
    @pl.when(cid == 0)
    def _():
        _part_copy(sid, lambda s: cnt_sh.at[s], lambda s: out_hbm.at[0].at[s])
    @pl.when(cid == 1)
    def _():
        _part_copy(sid, lambda s: cnt_sh.at[s], lambda s: out_hbm.at[1].at[s])


_deg = pl.kernel(
    _deg_body,
    out_type=jax.ShapeDtypeStruct((2, N, 16), jnp.float32),
    mesh=_mesh,
    scratch_types=[
        pltpu.VMEM_SHARED((N, 16), jnp.float32),
        pltpu.VMEM((K, 16), jnp.float32),
        pltpu.VMEM((DCHUNK, K), jnp.int32),
        pltpu.SemaphoreType.DMA,
    ],
)


def _agg_body(ha_hbm, hb_hbm, src_hbm, dst_hbm, out_hbm,
              acc_sh, schunk, dchunk, rows_a, rows_b, rows_c, rows_d, sem):
    cid = lax.axis_index("c")
    sid = lax.axis_index("s")
    bufs = (rows_a, rows_b, rows_c, rows_d)

    def run(h_hbm, out_slot):
        _part_copy(sid, lambda s: h_hbm.at[s], lambda s: acc_sh.at[s])
        plsc.subcore_barrier()
        base = sid * (ROWS // NSUB // CHUNK)
        @pl.loop(0, (ROWS // NSUB) // CHUNK)
        def _(ci):
            pltpu.sync_copy(src_hbm.at[base + ci], schunk)
            pltpu.sync_copy(dst_hbm.at[base + ci], dchunk)
            descs = [None] * CHUNK

            def gather(j):
                descs[j] = pltpu.async_copy(
                    h_hbm.at[schunk.at[j]], bufs[j % 4], sem)

            gather(0)
            gather(1)
            gather(2)
            for j in range(CHUNK):
                if j + 3 < CHUNK:
                    gather(j + 3)
                descs[j].wait()
                pltpu.sync_copy(bufs[j % 4], acc_sh.at[dchunk.at[j]],
                                add=True)
        plsc.subcore_barrier()
        _part_copy(sid, lambda s: acc_sh.at[s], lambda s: out_slot.at[s])

    @pl.when(cid == 0)
    def _():
        run(ha_hbm, out_hbm.at[0])
    @pl.when(cid == 1)
    def _():
        run(hb_hbm, out_hbm.at[1])


_agg = pl.kernel(
    _agg_body,
    out_type=jax.ShapeDtypeStruct((2, N, C), jnp.float32),
    mesh=_mesh,
    scratch_types=[
        pltpu.VMEM_SHARED((N, C), jnp.float32),
        pltpu.VMEM((CHUNK, K), jnp.int32),
        pltpu.VMEM((CHUNK, K), jnp.int32),
        pltpu.VMEM((K, C), jnp.float32),
        pltpu.VMEM((K, C), jnp.float32),
        pltpu.VMEM((K, C), jnp.float32),
        pltpu.VMEM((K, C), jnp.float32),
        pltpu.SemaphoreType.DMA,
    ],
)


def _dinv_of(p_ref):
    deg = p_ref[0, :, 0] + p_ref[1, :, 0] - 1.0
    return lax.rsqrt(deg)[:, None]


def _mm1_body(x_ref, w_ref, o_ref):
    o_ref[...] = jnp.dot(x_ref[...], w_ref[...],
                         preferred_element_type=jnp.float32)


def _scale_body(u_ref, p_ref, a_ref, b_ref):
    dinv = _dinv_of(p_ref)
    a_ref[...] = u_ref[:, :C] * dinv
    b_ref[...] = u_ref[:, C:] * dinv


def _mid_body(agg_ref, p_ref, b1_ref, wmu_ref, wls_ref, omu_ref, ols_ref):
    dinv = _dinv_of(p_ref)
    h = jnp.concatenate([agg_ref[0], agg_ref[1]], axis=1) * dinv + b1_ref[...]
    nrm = jnp.sqrt(jnp.sum(h * h, axis=1, keepdims=True))
    h = jnp.maximum(h / jnp.maximum(nrm, 1e-12), 0.0)
    omu_ref[...] = jnp.dot(h, wmu_ref[...],
                           preferred_element_type=jnp.float32) * dinv
    ols_ref[...] = jnp.dot(h, wls_ref[...],
                           preferred_element_type=jnp.float32) * dinv


def _fin_body(agg_ref, p_ref, bmu_ref, bls_ref, mu_ref, ls_ref):
    dinv = _dinv_of(p_ref)
    mu_ref[...] = agg_ref[0] * dinv + bmu_ref[...]
    ls_ref[...] = agg_ref[1] * dinv + bls_ref[...]


def _row_spec(width):
    return pl.BlockSpec((RB, width), lambda i: (i, 0))


_P_SPEC = pl.BlockSpec((2, RB, 16), lambda i: (0, i, 0))
_AGG_SPEC = pl.BlockSpec((2, RB, C), lambda i: (0, i, 0))


def _full(shape):
    return pl.BlockSpec(shape, lambda i: tuple(0 for _ in shape))


def kernel(x, edge_index, W1, b1, W_mu, b_mu, W_ls, b_ls):
    src = edge_index[0].astype(jnp.int32).reshape(ROWS // CHUNK, CHUNK, K)
    dst = edge_index[1].astype(jnp.int32).reshape(ROWS // CHUNK, CHUNK, K)
    dstd = edge_index[1].astype(jnp.int32).reshape(ROWS // DCHUNK, DCHUNK, K)
    ones16 = jnp.ones((RPS_LAST, 16), jnp.float32)
    grid = (N // RB,)
    f32 = jnp.float32

    p = _deg(dstd, ones16)
    u = pl.pallas_call(
        _mm1_body, grid=grid,
        in_specs=[_row_spec(C), _full((C, 2 * C))],
        out_specs=_row_spec(2 * C),
        out_shape=jax.ShapeDtypeStruct((N, 2 * C), f32))(x, W1)
    ha, hb = pl.pallas_call(
        _scale_body, grid=grid,
        in_specs=[_row_spec(2 * C), _P_SPEC],
        out_specs=[_row_spec(C), _row_spec(C)],
        out_shape=[jax.ShapeDtypeStruct((N, C), f32)] * 2)(u, p)
    agg1 = _agg(ha, hb, src, dst)
    hmu, hls = pl.pallas_call(
        _mid_body, grid=grid,
        in_specs=[_AGG_SPEC, _P_SPEC, _full((1, 2 * C)),
                  _full((2 * C, C)), _full((2 * C, C))],
        out_specs=[_row_spec(C), _row_spec(C)],
        out_shape=[jax.ShapeDtypeStruct((N, C), f32)] * 2)(
            agg1, p, b1.reshape(1, 2 * C), W_mu, W_ls)
    agg2 = _agg(hmu, hls, src, dst)
    mu, ls = pl.pallas_call(
        _fin_body, grid=grid,
        in_specs=[_AGG_SPEC, _P_SPEC, _full((1, C)), _full((1, C))],
        out_specs=[_row_spec(C), _row_spec(C)],
        out_shape=[jax.ShapeDtypeStruct((N, C), f32)] * 2)(
            agg2, p, b_mu.reshape(1, C), b_ls.reshape(1, C))
    return (mu, ls)


# speedup vs baseline: 2.6732x; 1.0172x over previous
import jax
import jax.numpy as jnp
from jax import lax
from jax.experimental import pallas as pl
from jax.experimental.pallas import tpu as pltpu
from jax.experimental.pallas import tpu_sc as plsc

N = 10000
E = 320000
C = 128
K = 80
CHUNK = 50
DCHUNK = 25
NSUB = 16
RPS = 624
RPS_LAST = N - RPS * (NSUB - 1)
ROWS = E // K
RB = 1000

_mesh = plsc.VectorSubcoreMesh(
    core_axis_name="c", subcore_axis_name="s", num_cores=2, num_subcores=NSUB)


def _part_copy(sid, mk_src, mk_dst):
    base = pl.multiple_of(sid * RPS, 8)
    @pl.when(sid < NSUB - 1)
    def _():
        pltpu.sync_copy(mk_src(pl.ds(base, RPS)), mk_dst(pl.ds(base, RPS)))
    @pl.when(sid == NSUB - 1)
    def _():
        last = pl.ds((NSUB - 1) * RPS, RPS_LAST)
        pltpu.sync_copy(mk_src(last), mk_dst(last))


def _deg_body(dst_hbm, ones_hbm, out_hbm, cnt_sh, ones_v, dchunk, sem):
    cid = lax.axis_index("c")
    sid = lax.axis_index("s")
    _part_copy(sid, lambda s: ones_hbm.at[pl.ds(0, s.size)], lambda s: cnt_sh.at[s])
    pltpu.sync_copy(ones_hbm.at[pl.ds(0, K)], ones_v)
    plsc.subcore_barrier()
    wchunk = (cid * NSUB + sid) * (ROWS // 32 // DCHUNK)
    @pl.loop(0, (ROWS // 32) // DCHUNK)
    def _(ci):
        pltpu.sync_copy(dst_hbm.at[wchunk + ci], dchunk)
        @pl.loop(0, DCHUNK)
        def _(j):
            pltpu.sync_copy(ones_v, cnt_sh.at[dchunk.at[j]], add=True)
    plsc.subcore_barrier()
    @pl.when(cid == 0)
    def _():
        _part_copy(sid, lambda s: cnt_sh.at[s], lambda s: out_hbm.at[0].at[s])
    @pl.when(cid == 1)
    def _():
        _part_copy(sid, lambda s: cnt_sh.at[s], lambda s: out_hbm.at[1].at[s])


_deg = pl.kernel(
    _deg_body,
    out_type=jax.ShapeDtypeStruct((2, N, 16), jnp.float32),
    mesh=_mesh,
    scratch_types=[
        pltpu.VMEM_SHARED((N, 16), jnp.float32),
        pltpu.VMEM((K, 16), jnp.float32),
        pltpu.VMEM((DCHUNK, K), jnp.int32),
        pltpu.SemaphoreType.DMA,
    ],
)


def _agg_body(ha_hbm, hb_hbm, src_hbm, dst_hbm, out_hbm,
              acc_sh, schunk, dchunk, rows_a, rows_b, rows_c, sem):
    cid = lax.axis_index("c")
    sid = lax.axis_index("s")
    bufs = (rows_a, rows_b, rows_c)

    def run(h_hbm, out_slot):
        _part_copy(sid, lambda s: h_hbm.at[s], lambda s: acc_sh.at[s])
        plsc.subcore_barrier()
        base = sid * (ROWS // NSUB // CHUNK)
        @pl.loop(0, (ROWS // NSUB) // CHUNK)
        def _(ci):
            pltpu.sync_copy(src_hbm.at[base + ci], schunk)
            pltpu.sync_copy(dst_hbm.at[base + ci], dchunk)
            descs = [None] * CHUNK

            def gather(j):
                descs[j] = pltpu.async_copy(
                    h_hbm.at[schunk.at[j]], bufs[j % 3], sem)

            gather(0)
            gather(1)
            for j in range(CHUNK):
                if j + 2 < CHUNK:
                    gather(j + 2)
                descs[j].wait()
                pltpu.sync_copy(bufs[j % 3], acc_sh.at[dchunk.at[j]],
                                add=True)
        plsc.subcore_barrier()
        _part_copy(sid, lambda s: acc_sh.at[s], lambda s: out_slot.at[s])

    @pl.when(cid == 0)
    def _():
        run(ha_hbm, out_hbm.at[0])
    @pl.when(cid == 1)
    def _():
        run(hb_hbm, out_hbm.at[1])


_agg = pl.kernel(
    _agg_body,
    out_type=jax.ShapeDtypeStruct((2, N, C), jnp.float32),
    mesh=_mesh,
    scratch_types=[
        pltpu.VMEM_SHARED((N, C), jnp.float32),
        pltpu.VMEM((CHUNK, K), jnp.int32),
        pltpu.VMEM((CHUNK, K), jnp.int32),
        pltpu.VMEM((K, C), jnp.float32),
        pltpu.VMEM((K, C), jnp.float32),
        pltpu.VMEM((K, C), jnp.float32),
        pltpu.SemaphoreType.DMA,
    ],
)


def _dinv_of(p_ref):
    deg = p_ref[0, :, 0] + p_ref[1, :, 0] - 1.0
    return lax.rsqrt(deg)[:, None]


def _mm1_body(x_ref, w_ref, o_ref):
    o_ref[...] = jnp.dot(x_ref[...], w_ref[...],
                         preferred_element_type=jnp.float32)


def _scale_body(u_ref, p_ref, a_ref, b_ref):
    dinv = _dinv_of(p_ref)
    a_ref[...] = u_ref[:, :C] * dinv
    b_ref[...] = u_ref[:, C:] * dinv


def _mid_body(agg_ref, p_ref, b1_ref, wmu_ref, wls_ref, omu_ref, ols_ref):
    dinv = _dinv_of(p_ref)
    h = jnp.concatenate([agg_ref[0], agg_ref[1]], axis=1) * dinv + b1_ref[...]
    nrm = jnp.sqrt(jnp.sum(h * h, axis=1, keepdims=True))
    h = jnp.maximum(h / jnp.maximum(nrm, 1e-12), 0.0)
    omu_ref[...] = jnp.dot(h, wmu_ref[...],
                           preferred_element_type=jnp.float32) * dinv
    ols_ref[...] = jnp.dot(h, wls_ref[...],
                           preferred_element_type=jnp.float32) * dinv


def _fin_body(agg_ref, p_ref, bmu_ref, bls_ref, mu_ref, ls_ref):
    dinv = _dinv_of(p_ref)
    mu_ref[...] = agg_ref[0] * dinv + bmu_ref[...]
    ls_ref[...] = agg_ref[1] * dinv + bls_ref[...]


def _row_spec(width):
    return pl.BlockSpec((RB, width), lambda i: (i, 0))


_P_SPEC = pl.BlockSpec((2, RB, 16), lambda i: (0, i, 0))
_AGG_SPEC = pl.BlockSpec((2, RB, C), lambda i: (0, i, 0))


def _full(shape):
    return pl.BlockSpec(shape, lambda i: tuple(0 for _ in shape))


def kernel(x, edge_index, W1, b1, W_mu, b_mu, W_ls, b_ls):
    src = edge_index[0].astype(jnp.int32).reshape(ROWS // CHUNK, CHUNK, K)
    dst = edge_index[1].astype(jnp.int32).reshape(ROWS // CHUNK, CHUNK, K)
    dstd = edge_index[1].astype(jnp.int32).reshape(ROWS // DCHUNK, DCHUNK, K)
    ones16 = jnp.ones((RPS_LAST, 16), jnp.float32)
    grid = (N // RB,)
    f32 = jnp.float32

    p = _deg(dstd, ones16)
    u = pl.pallas_call(
        _mm1_body, grid=grid,
        in_specs=[_row_spec(C), _full((C, 2 * C))],
        out_specs=_row_spec(2 * C),
        out_shape=jax.ShapeDtypeStruct((N, 2 * C), f32))(x, W1)
    ha, hb = pl.pallas_call(
        _scale_body, grid=grid,
        in_specs=[_row_spec(2 * C), _P_SPEC],
        out_specs=[_row_spec(C), _row_spec(C)],
        out_shape=[jax.ShapeDtypeStruct((N, C), f32)] * 2)(u, p)
    agg1 = _agg(ha, hb, src, dst)
    hmu, hls = pl.pallas_call(
        _mid_body, grid=grid,
        in_specs=[_AGG_SPEC, _P_SPEC, _full((1, 2 * C)),
                  _full((2 * C, C)), _full((2 * C, C))],
        out_specs=[_row_spec(C), _row_spec(C)],
        out_shape=[jax.ShapeDtypeStruct((N, C), f32)] * 2)(
            agg1, p, b1.reshape(1, 2 * C), W_mu, W_ls)
    agg2 = _agg(hmu, hls, src, dst)
    mu, ls = pl.pallas_call(
        _fin_body, grid=grid,
        in_specs=[_AGG_SPEC, _P_SPEC, _full((1, C)), _full((1, C))],
        out_specs=[_row_spec(C), _row_spec(C)],
        out_shape=[jax.ShapeDtypeStruct((N, C), f32)] * 2)(
            agg2, p, b_mu.reshape(1, C), b_ls.reshape(1, C))
    return (mu, ls)


# speedup vs baseline: 2.7139x; 1.0152x over previous
import jax
import jax.numpy as jnp
from jax import lax
from jax.experimental import pallas as pl
from jax.experimental.pallas import tpu as pltpu
from jax.experimental.pallas import tpu_sc as plsc

N = 10000
E = 320000
C = 128
K = 80
CHUNK = 50
DCHUNK = 25
NSUB = 16
RPS = 624
RPS_LAST = N - RPS * (NSUB - 1)
ROWS = E // K
RB = 1000

_mesh = plsc.VectorSubcoreMesh(
    core_axis_name="c", subcore_axis_name="s", num_cores=2, num_subcores=NSUB)


def _part_copy(sid, mk_src, mk_dst):
    base = pl.multiple_of(sid * RPS, 8)
    @pl.when(sid < NSUB - 1)
    def _():
        pltpu.sync_copy(mk_src(pl.ds(base, RPS)), mk_dst(pl.ds(base, RPS)))
    @pl.when(sid == NSUB - 1)
    def _():
        last = pl.ds((NSUB - 1) * RPS, RPS_LAST)
        pltpu.sync_copy(mk_src(last), mk_dst(last))


def _deg_body(dst_hbm, ones_hbm, out_hbm, cnt_sh, ones_v, dchunk, sem):
    cid = lax.axis_index("c")
    sid = lax.axis_index("s")
    _part_copy(sid, lambda s: ones_hbm.at[pl.ds(0, s.size)], lambda s: cnt_sh.at[s])
    pltpu.sync_copy(ones_hbm.at[pl.ds(0, K)], ones_v)
    plsc.subcore_barrier()
    wchunk = (cid * NSUB + sid) * (ROWS // 32 // DCHUNK)
    @pl.loop(0, (ROWS // 32) // DCHUNK)
    def _(ci):
        pltpu.sync_copy(dst_hbm.at[wchunk + ci], dchunk)
        descs = [
            pltpu.async_copy(ones_v, cnt_sh.at[dchunk.at[j]], sem, add=True)
            for j in range(DCHUNK)
        ]
        for d in descs:
            d.wait()
    plsc.subcore_barrier()
    @pl.when(cid == 0)
    def _():
        _part_copy(sid, lambda s: cnt_sh.at[s], lambda s: out_hbm.at[0].at[s])
    @pl.when(cid == 1)
    def _():
        _part_copy(sid, lambda s: cnt_sh.at[s], lambda s: out_hbm.at[1].at[s])


_deg = pl.kernel(
    _deg_body,
    out_type=jax.ShapeDtypeStruct((2, N, 16), jnp.float32),
    mesh=_mesh,
    scratch_types=[
        pltpu.VMEM_SHARED((N, 16), jnp.float32),
        pltpu.VMEM((K, 16), jnp.float32),
        pltpu.VMEM((DCHUNK, K), jnp.int32),
        pltpu.SemaphoreType.DMA,
    ],
)


def _agg_body(ha_hbm, hb_hbm, src_hbm, dst_hbm, out_hbm,
              acc_sh, schunk, dchunk, rows_a, rows_b, rows_c, sem):
    cid = lax.axis_index("c")
    sid = lax.axis_index("s")
    bufs = (rows_a, rows_b, rows_c)

    def run(h_hbm, out_slot):
        _part_copy(sid, lambda s: h_hbm.at[s], lambda s: acc_sh.at[s])
        plsc.subcore_barrier()
        base = sid * (ROWS // NSUB // CHUNK)
        @pl.loop(0, (ROWS // NSUB) // CHUNK)
        def _(ci):
            pltpu.sync_copy(src_hbm.at[base + ci], schunk)
            pltpu.sync_copy(dst_hbm.at[base + ci], dchunk)
            descs = [None] * CHUNK

            def gather(j):
                descs[j] = pltpu.async_copy(
                    h_hbm.at[schunk.at[j]], bufs[j % 3], sem)

            gather(0)
            gather(1)
            for j in range(CHUNK):
                if j + 2 < CHUNK:
                    gather(j + 2)
                descs[j].wait()
                pltpu.sync_copy(bufs[j % 3], acc_sh.at[dchunk.at[j]],
                                add=True)
        plsc.subcore_barrier()
        _part_copy(sid, lambda s: acc_sh.at[s], lambda s: out_slot.at[s])

    @pl.when(cid == 0)
    def _():
        run(ha_hbm, out_hbm.at[0])
    @pl.when(cid == 1)
    def _():
        run(hb_hbm, out_hbm.at[1])


_agg = pl.kernel(
    _agg_body,
    out_type=jax.ShapeDtypeStruct((2, N, C), jnp.float32),
    mesh=_mesh,
    scratch_types=[
        pltpu.VMEM_SHARED((N, C), jnp.float32),
        pltpu.VMEM((CHUNK, K), jnp.int32),
        pltpu.VMEM((CHUNK, K), jnp.int32),
        pltpu.VMEM((K, C), jnp.float32),
        pltpu.VMEM((K, C), jnp.float32),
        pltpu.VMEM((K, C), jnp.float32),
        pltpu.SemaphoreType.DMA,
    ],
)


def _dinv_of(p_ref):
    deg = p_ref[0, :, 0] + p_ref[1, :, 0] - 1.0
    return lax.rsqrt(deg)[:, None]


def _mm1_body(x_ref, w_ref, o_ref):
    o_ref[...] = jnp.dot(x_ref[...], w_ref[...],
                         preferred_element_type=jnp.float32)


def _scale_body(u_ref, p_ref, a_ref, b_ref):
    dinv = _dinv_of(p_ref)
    a_ref[...] = u_ref[:, :C] * dinv
    b_ref[...] = u_ref[:, C:] * dinv


def _mid_body(agg_ref, p_ref, b1_ref, wmu_ref, wls_ref, omu_ref, ols_ref):
    dinv = _dinv_of(p_ref)
    h = jnp.concatenate([agg_ref[0], agg_ref[1]], axis=1) * dinv + b1_ref[...]
    nrm = jnp.sqrt(jnp.sum(h * h, axis=1, keepdims=True))
    h = jnp.maximum(h / jnp.maximum(nrm, 1e-12), 0.0)
    omu_ref[...] = jnp.dot(h, wmu_ref[...],
                           preferred_element_type=jnp.float32) * dinv
    ols_ref[...] = jnp.dot(h, wls_ref[...],
                           preferred_element_type=jnp.float32) * dinv


def _fin_body(agg_ref, p_ref, bmu_ref, bls_ref, mu_ref, ls_ref):
    dinv = _dinv_of(p_ref)
    mu_ref[...] = agg_ref[0] * dinv + bmu_ref[...]
    ls_ref[...] = agg_ref[1] * dinv + bls_ref[...]


def _row_spec(width):
    return pl.BlockSpec((RB, width), lambda i: (i, 0))


_P_SPEC = pl.BlockSpec((2, RB, 16), lambda i: (0, i, 0))
_AGG_SPEC = pl.BlockSpec((2, RB, C), lambda i: (0, i, 0))


def _full(shape):
    return pl.BlockSpec(shape, lambda i: tuple(0 for _ in shape))


def kernel(x, edge_index, W1, b1, W_mu, b_mu, W_ls, b_ls):
    src = edge_index[0].astype(jnp.int32).reshape(ROWS // CHUNK, CHUNK, K)
    dst = edge_index[1].astype(jnp.int32).reshape(ROWS // CHUNK, CHUNK, K)
    dstd = edge_index[1].astype(jnp.int32).reshape(ROWS // DCHUNK, DCHUNK, K)
    ones16 = jnp.ones((RPS_LAST, 16), jnp.float32)
    grid = (N // RB,)
    f32 = jnp.float32

    p = _deg(dstd, ones16)
    u = pl.pallas_call(
        _mm1_body, grid=grid,
        in_specs=[_row_spec(C), _full((C, 2 * C))],
        out_specs=_row_spec(2 * C),
        out_shape=jax.ShapeDtypeStruct((N, 2 * C), f32))(x, W1)
    ha, hb = pl.pallas_call(
        _scale_body, grid=grid,
        in_specs=[_row_spec(2 * C), _P_SPEC],
        out_specs=[_row_spec(C), _row_spec(C)],
        out_shape=[jax.ShapeDtypeStruct((N, C), f32)] * 2)(u, p)
    agg1 = _agg(ha, hb, src, dst)
    hmu, hls = pl.pallas_call(
        _mid_body, grid=grid,
        in_specs=[_AGG_SPEC, _P_SPEC, _full((1, 2 * C)),
                  _full((2 * C, C)), _full((2 * C, C))],
        out_specs=[_row_spec(C), _row_spec(C)],
        out_shape=[jax.ShapeDtypeStruct((N, C), f32)] * 2)(
            agg1, p, b1.reshape(1, 2 * C), W_mu, W_ls)
    agg2 = _agg(hmu, hls, src, dst)
    mu, ls = pl.pallas_call(
        _fin_body, grid=grid,
        in_specs=[_AGG_SPEC, _P_SPEC, _full((1, C)), _full((1, C))],
        out_specs=[_row_spec(C), _row_spec(C)],
        out_shape=[jax.ShapeDtypeStruct((N, C), f32)] * 2)(
            agg2, p, b_mu.reshape(1, C), b_ls.reshape(1, C))
    return (mu, ls)
